# baseline (speedup 1.0000x reference)
import functools

import jax
import jax.numpy as jnp
from jax import lax
from jax.experimental import pallas as pl
from jax.experimental.pallas import tpu as pltpu
from jax.experimental.pallas import tpu_sc as plsc

VOCAB = 1000000
EMBED_DIM = 64
BATCH = 4096
SEQ = 200

NC = 2
NS = 16
NW = NC * NS

TOTAL = BATCH * SEQ
PER_W = TOTAL // NW
IDX_CHUNK = 128
G = 4
STAGE = G * IDX_CHUNK
N_STAGES = PER_W // STAGE
ROWS2D_PER_STAGE = G


def _make_kernel():
  mesh = plsc.VectorSubcoreMesh(core_axis_name="c", subcore_axis_name="s")

  @functools.partial(
      pl.kernel,
      out_type=jax.ShapeDtypeStruct((TOTAL, EMBED_DIM), jnp.float32),
      mesh=mesh,
      scratch_types=[
          pltpu.VMEM((G, IDX_CHUNK), jnp.int32),
          pltpu.VMEM((STAGE, EMBED_DIM), jnp.float32),
          pltpu.SemaphoreType.DMA,
      ],
      compiler_params=pltpu.CompilerParams(use_tc_tiling_on_sc=False),
  )
  def gather_kernel(table_hbm, idx_hbm, out_hbm, idx_v, rows_v, sem):
    wid = lax.axis_index("s") * NC + lax.axis_index("c")
    row0 = wid * (PER_W // IDX_CHUNK)

    @pl.loop(0, N_STAGES)
    def _stage(st):
      r = row0 + st * G
      pltpu.sync_copy(idx_hbm.at[pl.ds(r, G)], idx_v)
      copies = []
      for j in range(G):
        copies.append(
            pltpu.async_copy(
                table_hbm.at[idx_v.at[j]],
                rows_v.at[pl.ds(j * IDX_CHUNK, IDX_CHUNK)],
                sem,
            ))
      for c in copies:
        c.wait()
      pltpu.sync_copy(rows_v, out_hbm.at[pl.ds(r * IDX_CHUNK, STAGE)])

  return gather_kernel


_gather = _make_kernel()


@jax.jit
def kernel(input_ids, weight):
  idx2d = input_ids.reshape(TOTAL // IDX_CHUNK, IDX_CHUNK).astype(jnp.int32)
  out = _gather(weight, idx2d)
  return out.reshape(BATCH, SEQ, EMBED_DIM)


# speedup vs baseline: 1.0435x; 1.0435x over previous
import functools

import jax
import jax.numpy as jnp
from jax import lax
from jax.experimental import pallas as pl
from jax.experimental.pallas import tpu as pltpu
from jax.experimental.pallas import tpu_sc as plsc

VOCAB = 1000000
EMBED_DIM = 64
BATCH = 4096
SEQ = 200

NC = 2
NS = 16
NW = NC * NS

TOTAL = BATCH * SEQ
PER_W = TOTAL // NW
IDX_CHUNK = 128
G = 5
STAGE = G * IDX_CHUNK
N_STAGES = PER_W // STAGE
N_PAIRS = N_STAGES // 2


def _make_kernel():
  mesh = plsc.VectorSubcoreMesh(core_axis_name="c", subcore_axis_name="s")

  @functools.partial(
      pl.kernel,
      out_type=jax.ShapeDtypeStruct((TOTAL, EMBED_DIM), jnp.float32),
      mesh=mesh,
      scratch_types=[
          pltpu.VMEM((2, G, IDX_CHUNK), jnp.int32),
          pltpu.VMEM((2, STAGE, EMBED_DIM), jnp.float32),
          [pltpu.SemaphoreType.DMA] * 2,
          [pltpu.SemaphoreType.DMA] * 2,
          [pltpu.SemaphoreType.DMA] * 2,
      ],
      compiler_params=pltpu.CompilerParams(use_tc_tiling_on_sc=False),
  )
  def gather_kernel(table_hbm, idx_hbm, out_hbm, idx_v, rows_v, sem_i,
                    sem_g, sem_o):
    wid = lax.axis_index("s") * NC + lax.axis_index("c")
    row0 = wid * (PER_W // IDX_CHUNK)

    def idx_copy(st, b):
      return pltpu.make_async_copy(
          idx_hbm.at[pl.ds(row0 + st * G, G)], idx_v.at[b], sem_i[b])

    def gather_copy(b, j):
      return pltpu.make_async_copy(
          table_hbm.at[idx_v.at[b, j]],
          rows_v.at[b, pl.ds(j * IDX_CHUNK, IDX_CHUNK)],
          sem_g[b])

    def out_copy(st, b):
      return pltpu.make_async_copy(
          rows_v.at[b],
          out_hbm.at[pl.ds((row0 + st * G) * IDX_CHUNK, STAGE)],
          sem_o[b])

    def fire_gathers(b):
      for j in range(G):
        gather_copy(b, j).start()

    def drain_gathers(b):
      for j in range(G):
        gather_copy(b, j).wait()

    idx_copy(0, 0).start()
    idx_copy(1, 1).start()
    idx_copy(0, 0).wait()
    fire_gathers(0)

    drain_gathers(0)
    out_copy(0, 0).start()
    idx_copy(2, 0).start()
    idx_copy(1, 1).wait()
    fire_gathers(1)

    drain_gathers(1)
    out_copy(1, 1).start()
    idx_copy(3, 1).start()
    idx_copy(2, 0).wait()
    out_copy(0, 0).wait()
    fire_gathers(0)

    @pl.loop(1, N_PAIRS - 1)
    def _pair(p):
      s0 = p * 2
      for b in range(2):
        s = s0 + b
        nb = 1 - b
        drain_gathers(b)
        out_copy(s, b).start()
        idx_copy(s + 2, b).start()
        idx_copy(s + 1, nb).wait()
        out_copy(s - 1, nb).wait()
        fire_gathers(nb)

    drain_gathers(0)
    out_copy(N_STAGES - 2, 0).start()
    idx_copy(N_STAGES - 1, 1).wait()
    out_copy(N_STAGES - 3, 1).wait()
    fire_gathers(1)

    drain_gathers(1)
    out_copy(N_STAGES - 1, 1).start()
    out_copy(N_STAGES - 2, 0).wait()
    out_copy(N_STAGES - 1, 1).wait()

  return gather_kernel


_gather = _make_kernel()


@jax.jit
def kernel(input_ids, weight):
  idx2d = input_ids.reshape(TOTAL // IDX_CHUNK, IDX_CHUNK).astype(jnp.int32)
  out = _gather(weight, idx2d)
  return out.reshape(BATCH, SEQ, EMBED_DIM)


# speedup vs baseline: 1.0438x; 1.0003x over previous
import functools

import jax
import jax.numpy as jnp
from jax import lax
from jax.experimental import pallas as pl
from jax.experimental.pallas import tpu as pltpu
from jax.experimental.pallas import tpu_sc as plsc

VOCAB = 1000000
EMBED_DIM = 64
BATCH = 4096
SEQ = 200

NC = 2
NS = 16
NW = NC * NS
ROWS_PER_W = BATCH // NW
R = 4
N_STAGES = ROWS_PER_W // R
N_PAIRS = N_STAGES // 2
CHUNKS = ((0, 128), (128, 72))


def _make_kernel():
  mesh = plsc.VectorSubcoreMesh(core_axis_name="c", subcore_axis_name="s")

  @functools.partial(
      pl.kernel,
      out_type=jax.ShapeDtypeStruct((BATCH, SEQ, EMBED_DIM), jnp.float32),
      mesh=mesh,
      scratch_types=[
          pltpu.VMEM((2, R, SEQ), jnp.int32),
          pltpu.VMEM((2, R, SEQ, EMBED_DIM), jnp.float32),
          [pltpu.SemaphoreType.DMA] * 2,
          [pltpu.SemaphoreType.DMA] * 2,
          [pltpu.SemaphoreType.DMA] * 2,
      ],
      compiler_params=pltpu.CompilerParams(use_tc_tiling_on_sc=False),
  )
  def gather_kernel(table_hbm, idx_hbm, out_hbm, idx_v, rows_v, sem_i,
                    sem_g, sem_o):
    wid = lax.axis_index("s") * NC + lax.axis_index("c")
    row0 = wid * ROWS_PER_W

    def idx_copy(st, b):
      return pltpu.make_async_copy(
          idx_hbm.at[pl.ds(row0 + st * R, R)], idx_v.at[b], sem_i[b])

    def gather_copy(b, r, h):
      off, n = CHUNKS[h]
      return pltpu.make_async_copy(
          table_hbm.at[idx_v.at[b, r, pl.ds(off, n)]],
          rows_v.at[b, r, pl.ds(off, n)],
          sem_g[b])

    def out_copy(st, b):
      return pltpu.make_async_copy(
          rows_v.at[b], out_hbm.at[pl.ds(row0 + st * R, R)], sem_o[b])

    def fire_gathers(b):
      for r in range(R):
        for h in range(2):
          gather_copy(b, r, h).start()

    def drain_gathers(b):
      for r in range(R):
        for h in range(2):
          gather_copy(b, r, h).wait()

    idx_copy(0, 0).start()
    idx_copy(1, 1).start()
    idx_copy(0, 0).wait()
    fire_gathers(0)

    drain_gathers(0)
    out_copy(0, 0).start()
    idx_copy(2, 0).start()
    idx_copy(1, 1).wait()
    fire_gathers(1)

    drain_gathers(1)
    out_copy(1, 1).start()
    idx_copy(3, 1).start()
    idx_copy(2, 0).wait()
    out_copy(0, 0).wait()
    fire_gathers(0)

    @pl.loop(1, N_PAIRS - 1)
    def _pair(p):
      s0 = p * 2
      for b in range(2):
        s = s0 + b
        nb = 1 - b
        drain_gathers(b)
        out_copy(s, b).start()
        idx_copy(s + 2, b).start()
        idx_copy(s + 1, nb).wait()
        out_copy(s - 1, nb).wait()
        fire_gathers(nb)

    drain_gathers(0)
    out_copy(N_STAGES - 2, 0).start()
    idx_copy(N_STAGES - 1, 1).wait()
    out_copy(N_STAGES - 3, 1).wait()
    fire_gathers(1)

    drain_gathers(1)
    out_copy(N_STAGES - 1, 1).start()
    out_copy(N_STAGES - 2, 0).wait()
    out_copy(N_STAGES - 1, 1).wait()

  return gather_kernel


_gather = _make_kernel()


@jax.jit
def kernel(input_ids, weight):
  return _gather(weight, input_ids.astype(jnp.int32))


# speedup vs baseline: 1.0455x; 1.0016x over previous
import functools

import jax
import jax.numpy as jnp
from jax import lax
from jax.experimental import pallas as pl
from jax.experimental.pallas import tpu as pltpu
from jax.experimental.pallas import tpu_sc as plsc

VOCAB = 1000000
EMBED_DIM = 64
BATCH = 4096
SEQ = 200

NC = 2
NS = 16
NW = NC * NS
ROWS_PER_W = BATCH // NW
R = 4
N_STAGES = ROWS_PER_W // R
N_PAIRS = N_STAGES // 2
CHUNKS = ((0, 128), (128, 72))


def _make_kernel():
  mesh = plsc.VectorSubcoreMesh(core_axis_name="c", subcore_axis_name="s")

  @functools.partial(
      pl.kernel,
      out_type=jax.ShapeDtypeStruct((BATCH * SEQ, EMBED_DIM), jnp.float32),
      mesh=mesh,
      scratch_types=[
          pltpu.VMEM((2, R, SEQ), jnp.int32),
          pltpu.VMEM((2, R * SEQ, EMBED_DIM), jnp.float32),
          [pltpu.SemaphoreType.DMA] * 2,
          [pltpu.SemaphoreType.DMA] * 2,
          [pltpu.SemaphoreType.DMA] * 2,
      ],
      compiler_params=pltpu.CompilerParams(use_tc_tiling_on_sc=False),
  )
  def gather_kernel(table_hbm, idx_hbm, out_hbm, idx_v, rows_v, sem_i,
                    sem_g, sem_o):
    wid = lax.axis_index("s") * NC + lax.axis_index("c")
    row0 = wid * ROWS_PER_W

    def idx_copy(st, b):
      return pltpu.make_async_copy(
          idx_hbm.at[pl.ds(row0 + st * R, R)], idx_v.at[b], sem_i[b])

    def gather_copy(b, r, h):
      off, n = CHUNKS[h]
      return pltpu.make_async_copy(
          table_hbm.at[idx_v.at[b, r, pl.ds(off, n)]],
          rows_v.at[b, pl.ds(r * SEQ + off, n)],
          sem_g[b])

    def out_copy(st, b):
      return pltpu.make_async_copy(
          rows_v.at[b],
          out_hbm.at[pl.ds((row0 + st * R) * SEQ, R * SEQ)],
          sem_o[b])

    def fire_gathers(b):
      for r in range(R):
        for h in range(2):
          gather_copy(b, r, h).start()

    def drain_gathers(b):
      for r in range(R):
        for h in range(2):
          gather_copy(b, r, h).wait()

    idx_copy(0, 0).start()
    idx_copy(1, 1).start()
    idx_copy(0, 0).wait()
    fire_gathers(0)

    drain_gathers(0)
    out_copy(0, 0).start()
    idx_copy(2, 0).start()
    idx_copy(1, 1).wait()
    fire_gathers(1)

    drain_gathers(1)
    out_copy(1, 1).start()
    idx_copy(3, 1).start()
    idx_copy(2, 0).wait()
    out_copy(0, 0).wait()
    fire_gathers(0)

    @pl.loop(1, N_PAIRS - 1)
    def _pair(p):
      s0 = p * 2
      for b in range(2):
        s = s0 + b
        nb = 1 - b
        drain_gathers(b)
        out_copy(s, b).start()
        idx_copy(s + 2, b).start()
        idx_copy(s + 1, nb).wait()
        out_copy(s - 1, nb).wait()
        fire_gathers(nb)

    drain_gathers(0)
    out_copy(N_STAGES - 2, 0).start()
    idx_copy(N_STAGES - 1, 1).wait()
    out_copy(N_STAGES - 3, 1).wait()
    fire_gathers(1)

    drain_gathers(1)
    out_copy(N_STAGES - 1, 1).start()
    out_copy(N_STAGES - 2, 0).wait()
    out_copy(N_STAGES - 1, 1).wait()

  return gather_kernel


_gather = _make_kernel()


@jax.jit
def kernel(input_ids, weight):
  ids = jnp.minimum(input_ids.astype(jnp.int32), jnp.int32(VOCAB - 1))
  out = _gather(weight, ids)
  return out.reshape(BATCH, SEQ, EMBED_DIM)


# speedup vs baseline: 1.3884x; 1.3280x over previous
import functools

import jax
import jax.numpy as jnp
from jax import lax
from jax.experimental import pallas as pl
from jax.experimental.pallas import tpu as pltpu
from jax.experimental.pallas import tpu_sc as plsc

VOCAB = 1000000
EMBED_DIM = 64
BATCH = 4096
SEQ = 200

NC = 2
NS = 16
NW = NC * NS
ROWS_PER_W = BATCH // NW
R = 4
N_STAGES = ROWS_PER_W // R
N_PAIRS = N_STAGES // 2
CHUNKS = ((0, 128), (128, 72))


def _make_kernel():
  mesh = plsc.VectorSubcoreMesh(core_axis_name="c", subcore_axis_name="s")

  @functools.partial(
      pl.kernel,
      out_type=jax.ShapeDtypeStruct((BATCH * SEQ, 2 * EMBED_DIM), jnp.float32),
      mesh=mesh,
      scratch_types=[
          pltpu.VMEM((2, R, SEQ), jnp.int32),
          pltpu.VMEM((2, R * SEQ, EMBED_DIM), jnp.float32),
          [pltpu.SemaphoreType.DMA] * 2,
          [pltpu.SemaphoreType.DMA] * 2,
          [pltpu.SemaphoreType.DMA] * 2,
      ],
      compiler_params=pltpu.CompilerParams(use_tc_tiling_on_sc=False),
  )
  def gather_kernel(table_hbm, idx_hbm, out_hbm, idx_v, rows_v, sem_i,
                    sem_g, sem_o):
    wid = lax.axis_index("s") * NC + lax.axis_index("c")
    row0 = wid * ROWS_PER_W

    def idx_copy(st, b):
      return pltpu.make_async_copy(
          idx_hbm.at[pl.ds(row0 + st * R, R)], idx_v.at[b], sem_i[b])

    def gather_copy(b, r, h):
      off, n = CHUNKS[h]
      return pltpu.make_async_copy(
          table_hbm.at[idx_v.at[b, r, pl.ds(off, n)]],
          rows_v.at[b, pl.ds(r * SEQ + off, n)],
          sem_g[b])

    def out_copy(st, b):
      return pltpu.make_async_copy(
          rows_v.at[b],
          out_hbm.at[pl.ds((row0 + st * R) * SEQ, R * SEQ),
                     pl.ds(0, EMBED_DIM)],
          sem_o[b])

    def fire_gathers(b):
      for r in range(R):
        for h in range(2):
          gather_copy(b, r, h).start()

    def drain_gathers(b):
      for r in range(R):
        for h in range(2):
          gather_copy(b, r, h).wait()

    idx_copy(0, 0).start()
    idx_copy(1, 1).start()
    idx_copy(0, 0).wait()
    fire_gathers(0)

    drain_gathers(0)
    out_copy(0, 0).start()
    idx_copy(2, 0).start()
    idx_copy(1, 1).wait()
    fire_gathers(1)

    drain_gathers(1)
    out_copy(1, 1).start()
    idx_copy(3, 1).start()
    idx_copy(2, 0).wait()
    out_copy(0, 0).wait()
    fire_gathers(0)

    @pl.loop(1, N_PAIRS - 1)
    def _pair(p):
      s0 = p * 2
      for b in range(2):
        s = s0 + b
        nb = 1 - b
        drain_gathers(b)
        out_copy(s, b).start()
        idx_copy(s + 2, b).start()
        idx_copy(s + 1, nb).wait()
        out_copy(s - 1, nb).wait()
        fire_gathers(nb)

    drain_gathers(0)
    out_copy(N_STAGES - 2, 0).start()
    idx_copy(N_STAGES - 1, 1).wait()
    out_copy(N_STAGES - 3, 1).wait()
    fire_gathers(1)

    drain_gathers(1)
    out_copy(N_STAGES - 1, 1).start()
    out_copy(N_STAGES - 2, 0).wait()
    out_copy(N_STAGES - 1, 1).wait()

  return gather_kernel


_gather = _make_kernel()


@jax.jit
def kernel(input_ids, weight):
  ids = jnp.minimum(input_ids.astype(jnp.int32), jnp.int32(VOCAB - 1))
  out = _gather(weight, ids)
  return out[:, :EMBED_DIM].reshape(BATCH, SEQ, EMBED_DIM)
